# baseline (speedup 1.0000x reference)
import functools
import math

import jax
import jax.numpy as jnp
from jax.experimental import pallas as pl
from jax.experimental.pallas import tpu as pltpu
from jax.experimental.pallas import tpu_sc as plsc

B, S, D = 1, 2048, 768
E, TOP_K = 8, 2
DI = 2 * D
H = 4
DH = DI // H
K_CONV = 4
T = B * S
KT = TOP_K * T

TILE = 256
NT = KT // TILE + E
P = NT * TILE


def _router_body(x_ref, wg_ref, logits_ref, topi_ref, topw_ref):
    x = x_ref[...]
    logits = jnp.dot(x, wg_ref[...], preferred_element_type=jnp.float32)
    logits_ref[...] = logits
    idx = jax.lax.broadcasted_iota(jnp.int32, (T, E), 1)
    m1 = jnp.max(logits, axis=1, keepdims=True)
    a1 = jnp.min(jnp.where(logits == m1, idx, E), axis=1, keepdims=True)
    masked = jnp.where(idx == a1, -jnp.inf, logits)
    m2 = jnp.max(masked, axis=1, keepdims=True)
    a2 = jnp.min(jnp.where((logits == m2) & (idx != a1), idx, E), axis=1,
                 keepdims=True)
    topi_ref[...] = jnp.concatenate([a1, a2], axis=1)
    w0 = jax.nn.sigmoid(m1 - m2)
    topw_ref[...] = jnp.concatenate([w0, 1.0 - w0], axis=1)


def _router_call(x, W_gate):
    return pl.pallas_call(
        _router_body,
        out_shape=(
            jax.ShapeDtypeStruct((T, E), jnp.float32),
            jax.ShapeDtypeStruct((T, TOP_K), jnp.int32),
            jax.ShapeDtypeStruct((T, TOP_K), jnp.float32),
        ),
    )(x, W_gate)


_NW = 32


def _sc_gather(table, idx, nrows, chunk):
    dk = table.shape[1]
    per_w = nrows // _NW
    nchunks = per_w // chunk
    mesh = plsc.VectorSubcoreMesh(core_axis_name="c", subcore_axis_name="s")

    @functools.partial(
        pl.kernel, mesh=mesh,
        out_type=jax.ShapeDtypeStruct((nrows, dk), jnp.float32),
        scratch_types=[
            pltpu.VMEM((chunk,), jnp.int32),
            pltpu.VMEM((chunk, dk), jnp.float32),
            pltpu.SemaphoreType.DMA,
        ],
    )
    def k(table_hbm, idx_hbm, out_hbm, idx_v, rows_v, sem):
        wid = jax.lax.axis_index("s") * 2 + jax.lax.axis_index("c")
        base = wid * per_w
        for c in range(nchunks):
            off = base + c * chunk
            pltpu.sync_copy(idx_hbm.at[pl.ds(off, chunk)], idx_v)
            pltpu.async_copy(table_hbm.at[idx_v], rows_v, sem).wait()
            pltpu.sync_copy(rows_v, out_hbm.at[pl.ds(off, chunk)])

    return k(table, idx)


def _up_body(te_ref, xg_ref, lns_ref, lnb_ref, wup_ref, cw_ref, cb_ref,
             xm_ref, xc_ref, sz_ref):
    xg = xg_ref[...]
    mu = jnp.mean(xg, axis=1, keepdims=True)
    var = jnp.mean((xg - mu) * (xg - mu), axis=1, keepdims=True)
    xn = (xg - mu) / jnp.sqrt(var + 1e-5)
    xn = xn * lns_ref[0, 0] + lnb_ref[0, 0]
    up = jnp.dot(xn.astype(jnp.bfloat16), wup_ref[0],
                 preferred_element_type=jnp.float32)
    x_m = up[:, :DI]
    z = up[:, DI:]
    xc = jax.nn.silu(x_m * cw_ref[0, 0] + cb_ref[0, 0])
    xm_ref[...] = x_m.astype(jnp.bfloat16)
    xc_ref[...] = xc.astype(jnp.bfloat16)
    sz_ref[...] = (z * jax.nn.sigmoid(z)).astype(jnp.bfloat16)


def _up_call(xg, tile_expert, ln_scale, ln_bias, W_up, conv_w, conv_b):
    grid_spec = pltpu.PrefetchScalarGridSpec(
        num_scalar_prefetch=1,
        grid=(NT,),
        in_specs=[
            pl.BlockSpec((TILE, D), lambda j, te: (j, 0)),
            pl.BlockSpec((1, 1, D), lambda j, te: (te[j], 0, 0)),
            pl.BlockSpec((1, 1, D), lambda j, te: (te[j], 0, 0)),
            pl.BlockSpec((1, D, 2 * DI), lambda j, te: (te[j], 0, 0)),
            pl.BlockSpec((1, 1, DI), lambda j, te: (te[j], 0, 0)),
            pl.BlockSpec((1, 1, DI), lambda j, te: (te[j], 0, 0)),
        ],
        out_specs=[
            pl.BlockSpec((TILE, DI), lambda j, te: (j, 0)),
            pl.BlockSpec((TILE, DI), lambda j, te: (j, 0)),
            pl.BlockSpec((TILE, DI), lambda j, te: (j, 0)),
        ],
    )
    return pl.pallas_call(
        _up_body,
        grid_spec=grid_spec,
        out_shape=(
            jax.ShapeDtypeStruct((P, DI), jnp.bfloat16),
            jax.ShapeDtypeStruct((P, DI), jnp.bfloat16),
            jax.ShapeDtypeStruct((P, DI), jnp.bfloat16),
        ),
    )(tile_expert, xg, ln_scale.reshape(E, 1, D),
      ln_bias.reshape(E, 1, D), W_up.astype(jnp.bfloat16),
      conv_w[:, K_CONV - 1, :].reshape(E, 1, DI), conv_b.reshape(E, 1, DI))


def _cell_body(te_ref, xm_ref, xc_ref, sz_ref, wq_ref, wk_ref, wv_ref,
               wi_ref, bi_ref, skip_ref, mhs_ref, wd_ref, y_ref):
    xm16 = xm_ref[...]
    xc16 = xc_ref[...]
    q = jnp.dot(xc16, wq_ref[0], preferred_element_type=jnp.float32)
    k = jnp.dot(xc16, wk_ref[0], preferred_element_type=jnp.float32)
    v = jnp.dot(xm16, wv_ref[0], preferred_element_type=jnp.float32)
    xc = xc16.astype(jnp.float32)
    wi = wi_ref[0]
    q16 = q.astype(jnp.bfloat16)
    k16 = k.astype(jnp.bfloat16)
    v16 = v.astype(jnp.bfloat16)
    ipre = (jnp.dot(q16, wi[:DI], preferred_element_type=jnp.float32)
            + jnp.dot(k16, wi[DI:2 * DI], preferred_element_type=jnp.float32)
            + jnp.dot(v16, wi[2 * DI:], preferred_element_type=jnp.float32)
            + bi_ref[0, 0])
    inv_sqrt_dh = 1.0 / math.sqrt(DH)
    heads = []
    for h in range(H):
        qh = q[:, h * DH:(h + 1) * DH]
        kh = k[:, h * DH:(h + 1) * DH]
        vh = v[:, h * DH:(h + 1) * DH]
        qk = jnp.sum(qh * kh, axis=1, keepdims=True) * inv_sqrt_dh
        ih = ipre[:, h:h + 1]
        n = jnp.maximum(jnp.abs(qk), jnp.exp(-ih))
        hv = (qk / n) * vh
        hmu = jnp.mean(hv, axis=1, keepdims=True)
        hvar = jnp.mean((hv - hmu) * (hv - hmu), axis=1, keepdims=True)
        hn = (hv - hmu) / jnp.sqrt(hvar + 1e-5)
        heads.append(hn * mhs_ref[0, 0, h * DH:(h + 1) * DH])
    hn_all = jnp.concatenate(heads, axis=1)
    hs = hn_all + skip_ref[0, 0] * xc
    ho = hs * sz_ref[...].astype(jnp.float32)
    y_ref[...] = jnp.dot(ho.astype(jnp.bfloat16), wd_ref[0],
                         preferred_element_type=jnp.float32)


def _cell_call(xm, xc, sz, tile_expert, W_q, W_k, W_v, w_i, b_i, skip,
               mh_scale, W_down):
    grid_spec = pltpu.PrefetchScalarGridSpec(
        num_scalar_prefetch=1,
        grid=(NT,),
        in_specs=[
            pl.BlockSpec((TILE, DI), lambda j, te: (j, 0)),
            pl.BlockSpec((TILE, DI), lambda j, te: (j, 0)),
            pl.BlockSpec((TILE, DI), lambda j, te: (j, 0)),
            pl.BlockSpec((1, DI, DI), lambda j, te: (te[j], 0, 0)),
            pl.BlockSpec((1, DI, DI), lambda j, te: (te[j], 0, 0)),
            pl.BlockSpec((1, DI, DI), lambda j, te: (te[j], 0, 0)),
            pl.BlockSpec((1, 3 * DI, H), lambda j, te: (te[j], 0, 0)),
            pl.BlockSpec((1, 1, H), lambda j, te: (te[j], 0, 0)),
            pl.BlockSpec((1, 1, DI), lambda j, te: (te[j], 0, 0)),
            pl.BlockSpec((1, 1, DI), lambda j, te: (te[j], 0, 0)),
            pl.BlockSpec((1, DI, D), lambda j, te: (te[j], 0, 0)),
        ],
        out_specs=pl.BlockSpec((TILE, D), lambda j, te: (j, 0)),
    )
    bf = jnp.bfloat16
    return pl.pallas_call(
        _cell_body,
        grid_spec=grid_spec,
        out_shape=jax.ShapeDtypeStruct((P, D), jnp.float32),
    )(tile_expert, xm, xc, sz, W_q.astype(bf), W_k.astype(bf),
      W_v.astype(bf), w_i.astype(bf), b_i.reshape(E, 1, H),
      skip.reshape(E, 1, DI), mh_scale.reshape(E, 1, DI), W_down.astype(bf))


def _combine_body(x_ref, y0_ref, y1_ref, w_ref, o_ref):
    w0 = w_ref[:, 0:1]
    w1 = w_ref[:, 1:2]
    o_ref[...] = x_ref[...] + w0 * y0_ref[...] + w1 * y1_ref[...]


def _combine_call(x, yg, topw):
    return pl.pallas_call(
        _combine_body,
        grid=(T // TILE,),
        in_specs=[
            pl.BlockSpec((TILE, D), lambda j: (j, 0)),
            pl.BlockSpec((TILE, D), lambda j: (j, 0)),
            pl.BlockSpec((TILE, D), lambda j: (T // TILE + j, 0)),
            pl.BlockSpec((TILE, TOP_K), lambda j: (j, 0)),
        ],
        out_specs=pl.BlockSpec((TILE, D), lambda j: (j, 0)),
        out_shape=jax.ShapeDtypeStruct((T, D), jnp.float32),
    )(x, yg, yg, topw)


@jax.jit
def kernel(hidden_states, W_gate, ln_scale, ln_bias, W_up, conv_w, conv_b,
           W_q, W_k, W_v, w_i, b_i, w_f, b_f, skip, mh_scale, W_down):
    x = hidden_states.reshape(T, D)
    logits, topi, topw = _router_call(x, W_gate)

    flat_e = topi.reshape(-1)
    perm = jnp.argsort(flat_e, stable=True)
    sorted_e = flat_e[perm]
    counts = jnp.bincount(flat_e, length=E).astype(jnp.int32)
    tiles_pe = (counts + TILE - 1) // TILE
    cum_tiles = jnp.cumsum(tiles_pe)
    tiles_before = cum_tiles - tiles_pe
    offs = tiles_before * TILE
    cstart = jnp.cumsum(counts) - counts
    rank = jnp.arange(KT, dtype=jnp.int32) - cstart[sorted_e]
    dest = offs[sorted_e] + rank
    row_token = jnp.zeros((P,), jnp.int32).at[dest].set(
        (perm // TOP_K).astype(jnp.int32))
    pos = jnp.zeros((KT,), jnp.int32).at[perm].set(dest.astype(jnp.int32))
    pos2 = pos.reshape(T, TOP_K)
    pos_cat = jnp.concatenate([pos2[:, 0], pos2[:, 1]])
    tile_expert = jnp.minimum(
        jnp.searchsorted(cum_tiles, jnp.arange(NT, dtype=jnp.int32),
                         side='right'),
        E - 1).astype(jnp.int32)

    xg = _sc_gather(x, row_token, P, 96)
    xm, xc, sz = _up_call(xg, tile_expert, ln_scale, ln_bias, W_up,
                          conv_w, conv_b)
    y = _cell_call(xm, xc, sz, tile_expert, W_q, W_k, W_v, w_i, b_i, skip,
                   mh_scale, W_down)
    yg = _sc_gather(y, pos_cat, KT, 128)
    out = _combine_call(x, yg, topw)
    return out.reshape(B, S, D), logits


# speedup vs baseline: 1.0426x; 1.0426x over previous
import functools
import math

import jax
import jax.numpy as jnp
from jax.experimental import pallas as pl
from jax.experimental.pallas import tpu as pltpu
from jax.experimental.pallas import tpu_sc as plsc

B, S, D = 1, 2048, 768
E, TOP_K = 8, 2
DI = 2 * D
H = 4
DH = DI // H
K_CONV = 4
T = B * S
KT = TOP_K * T

TILE = 256
NT = KT // TILE + E
P = NT * TILE


def _router_body(x_ref, wg_ref, logits_ref, topi_ref, topw_ref):
    x = x_ref[...]
    logits = jnp.dot(x, wg_ref[...], preferred_element_type=jnp.float32)
    logits_ref[...] = logits
    idx = jax.lax.broadcasted_iota(jnp.int32, (T, E), 1)
    m1 = jnp.max(logits, axis=1, keepdims=True)
    a1 = jnp.min(jnp.where(logits == m1, idx, E), axis=1, keepdims=True)
    masked = jnp.where(idx == a1, -jnp.inf, logits)
    m2 = jnp.max(masked, axis=1, keepdims=True)
    a2 = jnp.min(jnp.where((logits == m2) & (idx != a1), idx, E), axis=1,
                 keepdims=True)
    topi_ref[...] = jnp.concatenate([a1, a2], axis=1)
    w0 = jax.nn.sigmoid(m1 - m2)
    topw_ref[...] = jnp.concatenate([w0, 1.0 - w0], axis=1)


def _router_call(x, W_gate):
    return pl.pallas_call(
        _router_body,
        out_shape=(
            jax.ShapeDtypeStruct((T, E), jnp.float32),
            jax.ShapeDtypeStruct((T, TOP_K), jnp.int32),
            jax.ShapeDtypeStruct((T, TOP_K), jnp.float32),
        ),
    )(x, W_gate)


_NW = 32


def _sc_gather(table, idx, nrows, chunk):
    dk = table.shape[1]
    per_w = nrows // _NW
    nchunks = per_w // chunk
    mesh = plsc.VectorSubcoreMesh(core_axis_name="c", subcore_axis_name="s")

    @functools.partial(
        pl.kernel, mesh=mesh,
        out_type=jax.ShapeDtypeStruct((nrows, dk), jnp.float32),
        scratch_types=[
            pltpu.VMEM((chunk,), jnp.int32),
            pltpu.VMEM((chunk, dk), jnp.float32),
            pltpu.SemaphoreType.DMA,
        ],
    )
    def k(table_hbm, idx_hbm, out_hbm, idx_v, rows_v, sem):
        wid = jax.lax.axis_index("s") * 2 + jax.lax.axis_index("c")
        base = wid * per_w
        for c in range(nchunks):
            off = base + c * chunk
            pltpu.sync_copy(idx_hbm.at[pl.ds(off, chunk)], idx_v)
            pltpu.async_copy(table_hbm.at[idx_v], rows_v, sem).wait()
            pltpu.sync_copy(rows_v, out_hbm.at[pl.ds(off, chunk)])

    return k(table, idx)


def _up_body(te_ref, xg_ref, lns_ref, lnb_ref, wup_ref, cw_ref, cb_ref,
             xm_ref, xc_ref, sz_ref):
    xg = xg_ref[...]
    mu = jnp.mean(xg, axis=1, keepdims=True)
    var = jnp.mean((xg - mu) * (xg - mu), axis=1, keepdims=True)
    xn = (xg - mu) / jnp.sqrt(var + 1e-5)
    xn = xn * lns_ref[0, 0] + lnb_ref[0, 0]
    up = jnp.dot(xn.astype(jnp.bfloat16), wup_ref[0],
                 preferred_element_type=jnp.float32)
    x_m = up[:, :DI]
    z = up[:, DI:]
    xc = jax.nn.silu(x_m * cw_ref[0, 0] + cb_ref[0, 0])
    xm_ref[...] = x_m.astype(jnp.bfloat16)
    xc_ref[...] = xc.astype(jnp.bfloat16)
    sz_ref[...] = (z * jax.nn.sigmoid(z)).astype(jnp.bfloat16)


def _up_call(xg, tile_expert, ln_scale, ln_bias, W_up, conv_w, conv_b):
    grid_spec = pltpu.PrefetchScalarGridSpec(
        num_scalar_prefetch=1,
        grid=(NT,),
        in_specs=[
            pl.BlockSpec((TILE, D), lambda j, te: (j, 0)),
            pl.BlockSpec((1, 1, D), lambda j, te: (te[j], 0, 0)),
            pl.BlockSpec((1, 1, D), lambda j, te: (te[j], 0, 0)),
            pl.BlockSpec((1, D, 2 * DI), lambda j, te: (te[j], 0, 0)),
            pl.BlockSpec((1, 1, DI), lambda j, te: (te[j], 0, 0)),
            pl.BlockSpec((1, 1, DI), lambda j, te: (te[j], 0, 0)),
        ],
        out_specs=[
            pl.BlockSpec((TILE, DI), lambda j, te: (j, 0)),
            pl.BlockSpec((TILE, DI), lambda j, te: (j, 0)),
            pl.BlockSpec((TILE, DI), lambda j, te: (j, 0)),
        ],
    )
    return pl.pallas_call(
        _up_body,
        grid_spec=grid_spec,
        out_shape=(
            jax.ShapeDtypeStruct((P, DI), jnp.bfloat16),
            jax.ShapeDtypeStruct((P, DI), jnp.bfloat16),
            jax.ShapeDtypeStruct((P, DI), jnp.bfloat16),
        ),
    )(tile_expert, xg, ln_scale.reshape(E, 1, D),
      ln_bias.reshape(E, 1, D), W_up.astype(jnp.bfloat16),
      conv_w[:, K_CONV - 1, :].reshape(E, 1, DI), conv_b.reshape(E, 1, DI))


def _cell_body(te_ref, xm_ref, xc_ref, sz_ref, wq_ref, wk_ref, wv_ref,
               wi_ref, bi_ref, skip_ref, mhs_ref, wd_ref, y_ref):
    xm16 = xm_ref[...]
    xc16 = xc_ref[...]
    q = jnp.dot(xc16, wq_ref[0], preferred_element_type=jnp.float32)
    k = jnp.dot(xc16, wk_ref[0], preferred_element_type=jnp.float32)
    v = jnp.dot(xm16, wv_ref[0], preferred_element_type=jnp.float32)
    xc = xc16.astype(jnp.float32)
    wi = wi_ref[0]
    q16 = q.astype(jnp.bfloat16)
    k16 = k.astype(jnp.bfloat16)
    v16 = v.astype(jnp.bfloat16)
    ipre = (jnp.dot(q16, wi[:DI], preferred_element_type=jnp.float32)
            + jnp.dot(k16, wi[DI:2 * DI], preferred_element_type=jnp.float32)
            + jnp.dot(v16, wi[2 * DI:], preferred_element_type=jnp.float32)
            + bi_ref[0, 0])
    inv_sqrt_dh = 1.0 / math.sqrt(DH)
    heads = []
    for h in range(H):
        qh = q[:, h * DH:(h + 1) * DH]
        kh = k[:, h * DH:(h + 1) * DH]
        vh = v[:, h * DH:(h + 1) * DH]
        qk = jnp.sum(qh * kh, axis=1, keepdims=True) * inv_sqrt_dh
        ih = ipre[:, h:h + 1]
        n = jnp.maximum(jnp.abs(qk), jnp.exp(-ih))
        hv = (qk / n) * vh
        hmu = jnp.mean(hv, axis=1, keepdims=True)
        hvar = jnp.mean((hv - hmu) * (hv - hmu), axis=1, keepdims=True)
        hn = (hv - hmu) / jnp.sqrt(hvar + 1e-5)
        heads.append(hn * mhs_ref[0, 0, h * DH:(h + 1) * DH])
    hn_all = jnp.concatenate(heads, axis=1)
    hs = hn_all + skip_ref[0, 0] * xc
    ho = hs * sz_ref[...].astype(jnp.float32)
    y_ref[...] = jnp.dot(ho.astype(jnp.bfloat16), wd_ref[0],
                         preferred_element_type=jnp.float32)


def _cell_call(xm, xc, sz, tile_expert, W_q, W_k, W_v, w_i, b_i, skip,
               mh_scale, W_down):
    grid_spec = pltpu.PrefetchScalarGridSpec(
        num_scalar_prefetch=1,
        grid=(NT,),
        in_specs=[
            pl.BlockSpec((TILE, DI), lambda j, te: (j, 0)),
            pl.BlockSpec((TILE, DI), lambda j, te: (j, 0)),
            pl.BlockSpec((TILE, DI), lambda j, te: (j, 0)),
            pl.BlockSpec((1, DI, DI), lambda j, te: (te[j], 0, 0)),
            pl.BlockSpec((1, DI, DI), lambda j, te: (te[j], 0, 0)),
            pl.BlockSpec((1, DI, DI), lambda j, te: (te[j], 0, 0)),
            pl.BlockSpec((1, 3 * DI, H), lambda j, te: (te[j], 0, 0)),
            pl.BlockSpec((1, 1, H), lambda j, te: (te[j], 0, 0)),
            pl.BlockSpec((1, 1, DI), lambda j, te: (te[j], 0, 0)),
            pl.BlockSpec((1, 1, DI), lambda j, te: (te[j], 0, 0)),
            pl.BlockSpec((1, DI, D), lambda j, te: (te[j], 0, 0)),
        ],
        out_specs=pl.BlockSpec((TILE, D), lambda j, te: (j, 0)),
    )
    bf = jnp.bfloat16
    return pl.pallas_call(
        _cell_body,
        grid_spec=grid_spec,
        out_shape=jax.ShapeDtypeStruct((P, D), jnp.float32),
    )(tile_expert, xm, xc, sz, W_q.astype(bf), W_k.astype(bf),
      W_v.astype(bf), w_i.astype(bf), b_i.reshape(E, 1, H),
      skip.reshape(E, 1, DI), mh_scale.reshape(E, 1, DI), W_down.astype(bf))


def _combine_body(x_ref, y0_ref, y1_ref, w_ref, o_ref):
    w0 = w_ref[:, 0:1]
    w1 = w_ref[:, 1:2]
    o_ref[...] = x_ref[...] + w0 * y0_ref[...] + w1 * y1_ref[...]


def _combine_call(x, yg, topw):
    return pl.pallas_call(
        _combine_body,
        grid=(T // TILE,),
        in_specs=[
            pl.BlockSpec((TILE, D), lambda j: (j, 0)),
            pl.BlockSpec((TILE, D), lambda j: (j, 0)),
            pl.BlockSpec((TILE, D), lambda j: (T // TILE + j, 0)),
            pl.BlockSpec((TILE, TOP_K), lambda j: (j, 0)),
        ],
        out_specs=pl.BlockSpec((TILE, D), lambda j: (j, 0)),
        out_shape=jax.ShapeDtypeStruct((T, D), jnp.float32),
    )(x, yg, yg, topw)


@jax.jit
def kernel(hidden_states, W_gate, ln_scale, ln_bias, W_up, conv_w, conv_b,
           W_q, W_k, W_v, w_i, b_i, w_f, b_f, skip, mh_scale, W_down):
    x = hidden_states.reshape(T, D)
    logits, topi, topw = _router_call(x, W_gate)

    flat_e = topi.reshape(-1)
    onehot = (flat_e[:, None]
              == jnp.arange(E, dtype=flat_e.dtype)[None, :]).astype(jnp.int32)
    csum = jnp.cumsum(onehot, axis=0)
    counts = csum[-1]
    tiles_pe = (counts + TILE - 1) // TILE
    cum_tiles = jnp.cumsum(tiles_pe)
    offs = (cum_tiles - tiles_pe) * TILE
    rank = jnp.take_along_axis(csum, flat_e[:, None], axis=1)[:, 0] - 1
    dest = (offs[flat_e] + rank).astype(jnp.int32)
    row_token = jnp.zeros((P,), jnp.int32).at[dest].set(
        jnp.arange(KT, dtype=jnp.int32) // TOP_K)
    pos2 = dest.reshape(T, TOP_K)
    pos_cat = jnp.concatenate([pos2[:, 0], pos2[:, 1]])
    tile_expert = jnp.minimum(
        jnp.searchsorted(cum_tiles, jnp.arange(NT, dtype=jnp.int32),
                         side='right'),
        E - 1).astype(jnp.int32)

    xg = _sc_gather(x, row_token, P, 96)
    xm, xc, sz = _up_call(xg, tile_expert, ln_scale, ln_bias, W_up,
                          conv_w, conv_b)
    y = _cell_call(xm, xc, sz, tile_expert, W_q, W_k, W_v, w_i, b_i, skip,
                   mh_scale, W_down)
    yg = _sc_gather(y, pos_cat, KT, 128)
    out = _combine_call(x, yg, topw)
    return out.reshape(B, S, D), logits


# speedup vs baseline: 1.0871x; 1.0426x over previous
import functools
import math

import jax
import jax.numpy as jnp
from jax.experimental import pallas as pl
from jax.experimental.pallas import tpu as pltpu
from jax.experimental.pallas import tpu_sc as plsc

B, S, D = 1, 2048, 768
E, TOP_K = 8, 2
DI = 2 * D
H = 4
DH = DI // H
K_CONV = 4
T = B * S
KT = TOP_K * T

TILE = 256
NT = KT // TILE + E
P = NT * TILE


def _router_body(x_ref, wg_ref, logits_ref, topi_ref, topw_ref):
    x = x_ref[...]
    logits = jnp.dot(x, wg_ref[...], preferred_element_type=jnp.float32)
    logits_ref[...] = logits
    idx = jax.lax.broadcasted_iota(jnp.int32, (T, E), 1)
    m1 = jnp.max(logits, axis=1, keepdims=True)
    a1 = jnp.min(jnp.where(logits == m1, idx, E), axis=1, keepdims=True)
    masked = jnp.where(idx == a1, -jnp.inf, logits)
    m2 = jnp.max(masked, axis=1, keepdims=True)
    a2 = jnp.min(jnp.where((logits == m2) & (idx != a1), idx, E), axis=1,
                 keepdims=True)
    topi_ref[...] = jnp.concatenate([a1, a2], axis=1)
    w0 = jax.nn.sigmoid(m1 - m2)
    topw_ref[...] = jnp.concatenate([w0, 1.0 - w0], axis=1)


def _router_call(x, W_gate):
    return pl.pallas_call(
        _router_body,
        out_shape=(
            jax.ShapeDtypeStruct((T, E), jnp.float32),
            jax.ShapeDtypeStruct((T, TOP_K), jnp.int32),
            jax.ShapeDtypeStruct((T, TOP_K), jnp.float32),
        ),
    )(x, W_gate)


_NW = 32


def _sc_gather(table, idx, nrows, chunk):
    dk = table.shape[1]
    per_w = nrows // _NW
    nchunks = per_w // chunk
    mesh = plsc.VectorSubcoreMesh(core_axis_name="c", subcore_axis_name="s")

    @functools.partial(
        pl.kernel, mesh=mesh,
        out_type=jax.ShapeDtypeStruct((nrows, dk), jnp.float32),
        scratch_types=[
            pltpu.VMEM((chunk,), jnp.int32),
            pltpu.VMEM((chunk, dk), jnp.float32),
            pltpu.SemaphoreType.DMA,
        ],
    )
    def k(table_hbm, idx_hbm, out_hbm, idx_v, rows_v, sem):
        wid = jax.lax.axis_index("s") * 2 + jax.lax.axis_index("c")
        base = wid * per_w
        for c in range(nchunks):
            off = base + c * chunk
            pltpu.sync_copy(idx_hbm.at[pl.ds(off, chunk)], idx_v)
            pltpu.async_copy(table_hbm.at[idx_v], rows_v, sem).wait()
            pltpu.sync_copy(rows_v, out_hbm.at[pl.ds(off, chunk)])

    return k(table, idx)


def _mega_body(te_ref, xg_ref, lns_ref, lnb_ref, wup_ref, cw_ref, cb_ref,
               wq_ref, wk_ref, wv_ref, wi_ref, bi_ref, skip_ref, mhs_ref,
               wd_ref, y_ref):
    xg = xg_ref[...]
    mu = jnp.mean(xg, axis=1, keepdims=True)
    var = jnp.mean((xg - mu) * (xg - mu), axis=1, keepdims=True)
    xn = (xg - mu) / jnp.sqrt(var + 1e-5)
    xn = xn * lns_ref[0, 0] + lnb_ref[0, 0]
    up = jnp.dot(xn.astype(jnp.bfloat16), wup_ref[0],
                 preferred_element_type=jnp.float32)
    x_m = up[:, :DI]
    z = up[:, DI:]
    xc = jax.nn.silu(x_m * cw_ref[0, 0] + cb_ref[0, 0])
    sz = z * jax.nn.sigmoid(z)
    xc16 = xc.astype(jnp.bfloat16)
    xm16 = x_m.astype(jnp.bfloat16)
    q = jnp.dot(xc16, wq_ref[0], preferred_element_type=jnp.float32)
    k = jnp.dot(xc16, wk_ref[0], preferred_element_type=jnp.float32)
    v = jnp.dot(xm16, wv_ref[0], preferred_element_type=jnp.float32)
    wi = wi_ref[0]
    q16 = q.astype(jnp.bfloat16)
    k16 = k.astype(jnp.bfloat16)
    v16 = v.astype(jnp.bfloat16)
    ipre = (jnp.dot(q16, wi[:DI], preferred_element_type=jnp.float32)
            + jnp.dot(k16, wi[DI:2 * DI], preferred_element_type=jnp.float32)
            + jnp.dot(v16, wi[2 * DI:], preferred_element_type=jnp.float32)
            + bi_ref[0, 0])
    inv_sqrt_dh = 1.0 / math.sqrt(DH)
    heads = []
    for h in range(H):
        qh = q[:, h * DH:(h + 1) * DH]
        kh = k[:, h * DH:(h + 1) * DH]
        vh = v[:, h * DH:(h + 1) * DH]
        qk = jnp.sum(qh * kh, axis=1, keepdims=True) * inv_sqrt_dh
        ih = ipre[:, h:h + 1]
        n = jnp.maximum(jnp.abs(qk), jnp.exp(-ih))
        hv = (qk / n) * vh
        hmu = jnp.mean(hv, axis=1, keepdims=True)
        hvar = jnp.mean((hv - hmu) * (hv - hmu), axis=1, keepdims=True)
        hn = (hv - hmu) / jnp.sqrt(hvar + 1e-5)
        heads.append(hn * mhs_ref[0, 0, h * DH:(h + 1) * DH])
    hn_all = jnp.concatenate(heads, axis=1)
    hs = hn_all + skip_ref[0, 0] * xc
    ho = hs * sz
    y_ref[...] = jnp.dot(ho.astype(jnp.bfloat16), wd_ref[0],
                         preferred_element_type=jnp.float32)


def _mega_call(xg, tile_expert, ln_scale, ln_bias, W_up, conv_w, conv_b,
               W_q, W_k, W_v, w_i, b_i, skip, mh_scale, W_down):
    grid_spec = pltpu.PrefetchScalarGridSpec(
        num_scalar_prefetch=1,
        grid=(NT,),
        in_specs=[
            pl.BlockSpec((TILE, D), lambda j, te: (j, 0)),
            pl.BlockSpec((1, 1, D), lambda j, te: (te[j], 0, 0)),
            pl.BlockSpec((1, 1, D), lambda j, te: (te[j], 0, 0)),
            pl.BlockSpec((1, D, 2 * DI), lambda j, te: (te[j], 0, 0)),
            pl.BlockSpec((1, 1, DI), lambda j, te: (te[j], 0, 0)),
            pl.BlockSpec((1, 1, DI), lambda j, te: (te[j], 0, 0)),
            pl.BlockSpec((1, DI, DI), lambda j, te: (te[j], 0, 0)),
            pl.BlockSpec((1, DI, DI), lambda j, te: (te[j], 0, 0)),
            pl.BlockSpec((1, DI, DI), lambda j, te: (te[j], 0, 0)),
            pl.BlockSpec((1, 3 * DI, H), lambda j, te: (te[j], 0, 0)),
            pl.BlockSpec((1, 1, H), lambda j, te: (te[j], 0, 0)),
            pl.BlockSpec((1, 1, DI), lambda j, te: (te[j], 0, 0)),
            pl.BlockSpec((1, 1, DI), lambda j, te: (te[j], 0, 0)),
            pl.BlockSpec((1, DI, D), lambda j, te: (te[j], 0, 0)),
        ],
        out_specs=pl.BlockSpec((TILE, D), lambda j, te: (j, 0)),
    )
    bf = jnp.bfloat16
    return pl.pallas_call(
        _mega_body,
        grid_spec=grid_spec,
        out_shape=jax.ShapeDtypeStruct((P, D), jnp.float32),
    )(tile_expert, xg, ln_scale.reshape(E, 1, D), ln_bias.reshape(E, 1, D),
      W_up.astype(bf), conv_w[:, K_CONV - 1, :].reshape(E, 1, DI),
      conv_b.reshape(E, 1, DI), W_q.astype(bf), W_k.astype(bf),
      W_v.astype(bf), w_i.astype(bf), b_i.reshape(E, 1, H),
      skip.reshape(E, 1, DI), mh_scale.reshape(E, 1, DI), W_down.astype(bf))


def _combine_body(x_ref, y0_ref, y1_ref, w_ref, o_ref):
    w0 = w_ref[:, 0:1]
    w1 = w_ref[:, 1:2]
    o_ref[...] = x_ref[...] + w0 * y0_ref[...] + w1 * y1_ref[...]


def _combine_call(x, yg, topw):
    return pl.pallas_call(
        _combine_body,
        grid=(T // TILE,),
        in_specs=[
            pl.BlockSpec((TILE, D), lambda j: (j, 0)),
            pl.BlockSpec((TILE, D), lambda j: (j, 0)),
            pl.BlockSpec((TILE, D), lambda j: (T // TILE + j, 0)),
            pl.BlockSpec((TILE, TOP_K), lambda j: (j, 0)),
        ],
        out_specs=pl.BlockSpec((TILE, D), lambda j: (j, 0)),
        out_shape=jax.ShapeDtypeStruct((T, D), jnp.float32),
    )(x, yg, yg, topw)


@jax.jit
def kernel(hidden_states, W_gate, ln_scale, ln_bias, W_up, conv_w, conv_b,
           W_q, W_k, W_v, w_i, b_i, w_f, b_f, skip, mh_scale, W_down):
    x = hidden_states.reshape(T, D)
    logits, topi, topw = _router_call(x, W_gate)

    flat_e = topi.reshape(-1)
    onehot = (flat_e[:, None]
              == jnp.arange(E, dtype=flat_e.dtype)[None, :]).astype(jnp.int32)
    csum = jnp.cumsum(onehot, axis=0)
    counts = csum[-1]
    tiles_pe = (counts + TILE - 1) // TILE
    cum_tiles = jnp.cumsum(tiles_pe)
    offs = (cum_tiles - tiles_pe) * TILE
    rank = jnp.take_along_axis(csum, flat_e[:, None], axis=1)[:, 0] - 1
    dest = (offs[flat_e] + rank).astype(jnp.int32)
    row_token = jnp.zeros((P,), jnp.int32).at[dest].set(
        jnp.arange(KT, dtype=jnp.int32) // TOP_K)
    pos2 = dest.reshape(T, TOP_K)
    pos_cat = jnp.concatenate([pos2[:, 0], pos2[:, 1]])
    tile_expert = jnp.minimum(
        jnp.searchsorted(cum_tiles, jnp.arange(NT, dtype=jnp.int32),
                         side='right'),
        E - 1).astype(jnp.int32)

    xg = _sc_gather(x, row_token, P, 96)
    y = _mega_call(xg, tile_expert, ln_scale, ln_bias, W_up, conv_w, conv_b,
                   W_q, W_k, W_v, w_i, b_i, skip, mh_scale, W_down)
    yg = _sc_gather(y, pos_cat, KT, 128)
    out = _combine_call(x, yg, topw)
    return out.reshape(B, S, D), logits
